# baseline (speedup 1.0000x reference)
import functools

import jax
import jax.numpy as jnp
from jax import lax
from jax.experimental import pallas as pl
from jax.experimental.pallas import tpu as pltpu
from jax.experimental.pallas import tpu_sc as plsc

N = 10000
E = 320000
D = 128
NC, NS, L = 2, 16, 16
NW = NC * NS
CH = 128
EPW = 10240
HC = 40
EP = EPW * NW
NP = 10240
RPT = NP // NS

_mesh = plsc.VectorSubcoreMesh(
    core_axis_name="c", subcore_axis_name="s", num_cores=NC, num_subcores=NS)


CPW = EPW // CH


@functools.partial(
    pl.kernel,
    out_type=jax.ShapeDtypeStruct((NC, NP), jnp.float32),
    mesh=_mesh,
    scratch_types=[
        pltpu.VMEM((CPW, CH), jnp.int32),
        pltpu.VMEM((CH,), jnp.float32),
        pltpu.VMEM((RPT,), jnp.float32),
        pltpu.VMEM_SHARED((NP,), jnp.float32),
        pltpu.SemaphoreType.DMA,
    ],
)
def _deg_kernel(dst_hbm, out_hbm, dstall, ones_v, z_v, deg_sh, sem):
    c = lax.axis_index("c")
    s = lax.axis_index("s")
    w = s * NC + c

    def _ones_init(i, carry):
        ones_v[pl.ds(i * L, L)] = jnp.full((L,), 1.0, jnp.float32)
        return carry

    lax.fori_loop(0, CH // L, _ones_init, 0)

    def _zero_init(i, carry):
        z_v[pl.ds(i * L, L)] = jnp.zeros((L,), jnp.float32)
        return carry

    lax.fori_loop(0, RPT // L, _zero_init, 0)
    pltpu.sync_copy(z_v, deg_sh.at[pl.ds(s * RPT, RPT)])
    pltpu.sync_copy(dst_hbm.at[pl.ds(w * CPW, CPW)], dstall)
    plsc.subcore_barrier()

    def _issue(t, carry):
        pltpu.async_copy(ones_v, deg_sh.at[dstall.at[t]], sem, add=True)
        return carry

    lax.fori_loop(0, CPW, _issue, 0)

    def _drain(t, carry):
        pltpu.make_async_copy(ones_v, deg_sh.at[dstall.at[t]], sem).wait()
        return carry

    lax.fori_loop(0, CPW, _drain, 0)
    plsc.subcore_barrier()
    pltpu.sync_copy(deg_sh.at[pl.ds(s * RPT, RPT)],
                    out_hbm.at[c, pl.ds(s * RPT, RPT)])


@functools.partial(
    pl.kernel,
    out_type=jax.ShapeDtypeStruct((NC, NP, D), jnp.float32),
    mesh=_mesh,
    scratch_types=[
        pltpu.VMEM((HC, CH), jnp.int32),
        pltpu.VMEM((HC, CH), jnp.int32),
        pltpu.VMEM((CH, D), jnp.float32),
        pltpu.VMEM((CH, D), jnp.float32),
        pltpu.VMEM_SHARED((NP, D), jnp.float32),
        pltpu.SemaphoreType.DMA,
        pltpu.SemaphoreType.DMA,
        pltpu.SemaphoreType.DMA,
        pltpu.SemaphoreType.DMA,
    ],
)
def _scatter_kernel(y_hbm, src_hbm, dst_hbm, z_hbm, out_hbm,
                    srcall, dstall, rows0, rows1, acc_sh,
                    sg0, sg1, ss0, ss1):
    c = lax.axis_index("c")
    s = lax.axis_index("s")
    w = s * NC + c
    rows = (rows0, rows1)
    sg = (sg0, sg1)
    ss = (ss0, ss1)

    zd = pltpu.async_copy(z_hbm.at[pl.ds(s * RPT, RPT)],
                          acc_sh.at[pl.ds(s * RPT, RPT)], sg0)
    sd = pltpu.async_copy(src_hbm.at[pl.ds(w * CPW, HC)], srcall, sg1)
    dd = pltpu.async_copy(dst_hbm.at[pl.ds(w * CPW, HC)], dstall, ss0)
    zd.wait()
    sd.wait()
    dd.wait()

    def _offset_src():
        def _o(i, carry):
            r = i // (CH // L)
            col = (i % (CH // L)) * L
            srcall[r, pl.ds(col, L)] = srcall[r, pl.ds(col, L)] + c * N
            return carry

        lax.fori_loop(0, HC * CH // L, _o, 0)

    _offset_src()

    def _issue_gather(t, b):
        pltpu.async_copy(y_hbm.at[srcall.at[t]], rows[b], sg[b])

    def _wait_gather(t, b):
        pltpu.make_async_copy(y_hbm.at[srcall.at[t]], rows[b], sg[b]).wait()

    def _issue_scatter(t, b):
        pltpu.async_copy(rows[b], acc_sh.at[dstall.at[t]], ss[b], add=True)

    def _wait_scatter(t, b):
        pltpu.make_async_copy(rows[b], acc_sh.at[dstall.at[t]], ss[b]).wait()

    def _run_half():
        _issue_gather(0, 0)
        _issue_gather(1, 1)
        _wait_gather(0, 0)
        _issue_scatter(0, 0)

        def _steady(i, carry):
            for u in (0, 1):
                t = 1 + i * 2 + u
                b = (1 + u) % 2
                _wait_scatter(t - 1, 1 - b)
                _issue_gather(t + 1, 1 - b)
                _wait_gather(t, b)
                _issue_scatter(t, b)
            return carry

        lax.fori_loop(0, (HC - 2) // 2, _steady, 0)
        t = HC - 1
        _wait_scatter(t - 1, 0)
        _wait_gather(t, 1)
        _issue_scatter(t, 1)
        _wait_scatter(t, 1)

    plsc.subcore_barrier()
    _run_half()
    base = w * CPW + HC
    pltpu.sync_copy(src_hbm.at[pl.ds(base, HC)], srcall)
    pltpu.sync_copy(dst_hbm.at[pl.ds(base, HC)], dstall)
    _offset_src()
    _run_half()
    plsc.subcore_barrier()
    pltpu.sync_copy(acc_sh.at[pl.ds(s * RPT, RPT)],
                    out_hbm.at[c, pl.ds(s * RPT, RPT)])


_BLK = 1000


def _l1a_body(x_ref, w_ref, u_ref):
    u_ref[...] = jnp.dot(x_ref[...], w_ref[...],
                         preferred_element_type=jnp.float32)


def _l1b_body(u_ref, d0_ref, d1_ref, y_ref, dinv_ref):
    dinv = lax.rsqrt(d0_ref[...] + d1_ref[...] + 1.0)
    y_ref[...] = u_ref[...] * dinv
    dinv_ref[...] = dinv


def _l2_body(p_ref, y_ref, dinv_ref, b_ref, w_ref, y2_ref):
    dinv = dinv_ref[...]
    h = jnp.maximum(dinv * (p_ref[0] + p_ref[1] + y_ref[...]) + b_ref[...], 0.0)
    y2_ref[...] = jnp.dot(h, w_ref[...],
                          preferred_element_type=jnp.float32) * dinv


def _l3_body(p_ref, y_ref, dinv_ref, b_ref, out_ref):
    out_ref[...] = jnp.maximum(
        dinv_ref[...] * (p_ref[0] + p_ref[1] + y_ref[...]) + b_ref[...], 0.0)


def _tc_l1a(x, W1):
    return pl.pallas_call(
        _l1a_body,
        grid=(N // _BLK,),
        in_specs=[
            pl.BlockSpec((_BLK, D), lambda i: (i, 0)),
            pl.BlockSpec((D, D), lambda i: (0, 0)),
        ],
        out_specs=pl.BlockSpec((_BLK, D), lambda i: (i, 0)),
        out_shape=jax.ShapeDtypeStruct((N, D), jnp.float32),
    )(x, W1)


def _tc_l1b(u, d0, d1):
    return pl.pallas_call(
        _l1b_body,
        grid=(N // _BLK,),
        in_specs=[
            pl.BlockSpec((_BLK, D), lambda i: (i, 0)),
            pl.BlockSpec((_BLK, 1), lambda i: (i, 0)),
            pl.BlockSpec((_BLK, 1), lambda i: (i, 0)),
        ],
        out_specs=[
            pl.BlockSpec((_BLK, D), lambda i: (i, 0)),
            pl.BlockSpec((_BLK, 1), lambda i: (i, 0)),
        ],
        out_shape=[
            jax.ShapeDtypeStruct((N, D), jnp.float32),
            jax.ShapeDtypeStruct((N, 1), jnp.float32),
        ],
    )(u, d0, d1)


def _tc_l2(p, y, dinv, b, W2):
    return pl.pallas_call(
        _l2_body,
        grid=(N // _BLK,),
        in_specs=[
            pl.BlockSpec((NC, _BLK, D), lambda i: (0, i, 0)),
            pl.BlockSpec((_BLK, D), lambda i: (i, 0)),
            pl.BlockSpec((_BLK, 1), lambda i: (i, 0)),
            pl.BlockSpec((1, D), lambda i: (0, 0)),
            pl.BlockSpec((D, D), lambda i: (0, 0)),
        ],
        out_specs=pl.BlockSpec((_BLK, D), lambda i: (i, 0)),
        out_shape=jax.ShapeDtypeStruct((N, D), jnp.float32),
    )(p, y, dinv, b, W2)


def _tc_l3(p, y, dinv, b):
    return pl.pallas_call(
        _l3_body,
        grid=(N // _BLK,),
        in_specs=[
            pl.BlockSpec((NC, _BLK, D), lambda i: (0, i, 0)),
            pl.BlockSpec((_BLK, D), lambda i: (i, 0)),
            pl.BlockSpec((_BLK, 1), lambda i: (i, 0)),
            pl.BlockSpec((1, D), lambda i: (0, 0)),
        ],
        out_specs=pl.BlockSpec((_BLK, D), lambda i: (i, 0)),
        out_shape=jax.ShapeDtypeStruct((N, D), jnp.float32),
    )(p, y, dinv, b)


def kernel(x, edge_index, W1, b1, W2, b2):
    src = edge_index[0].astype(jnp.int32)
    dst = edge_index[1].astype(jnp.int32)
    ppw = EPW - E // NW
    dummy_src = jnp.tile((jnp.arange(ppw, dtype=jnp.int32) * 41) % N, (NW, 1))
    dummy_dst = jnp.tile(N + (jnp.arange(ppw, dtype=jnp.int32) % (NP - N)),
                         (NW, 1))
    src_p = jnp.concatenate(
        [src.reshape(NW, E // NW), dummy_src], axis=1).reshape(EP)
    dst_p = jnp.concatenate(
        [dst.reshape(NW, E // NW), dummy_dst], axis=1).reshape(EP)

    src2d = src_p.reshape(EP // CH, CH)
    dst2d = dst_p.reshape(EP // CH, CH)

    degp = _deg_kernel(dst2d)
    d0 = degp[0, :N, None]
    d1 = degp[1, :N, None]

    zrows = jnp.zeros((NP, D), jnp.float32)

    u1 = _tc_l1a(x, W1)
    y1, dinv = _tc_l1b(u1, d0, d1)
    y1d = jnp.concatenate([y1, y1], axis=0)
    p1 = _scatter_kernel(y1d, src2d, dst2d, zrows)
    y2 = _tc_l2(p1, y1, dinv, b1.reshape(1, D), W2)
    y2d = jnp.concatenate([y2, y2], axis=0)
    p2 = _scatter_kernel(y2d, src2d, dst2d, zrows)
    return _tc_l3(p2, y2, dinv, b2.reshape(1, D))


# speedup vs baseline: 1.0814x; 1.0814x over previous
import functools

import jax
import jax.numpy as jnp
from jax import lax
from jax.experimental import pallas as pl
from jax.experimental.pallas import tpu as pltpu
from jax.experimental.pallas import tpu_sc as plsc

N = 10000
E = 320000
D = 128
NC, NS, L = 2, 16, 16
NW = NC * NS
CH = 128
EPW = 10240
HC = 40
EP = EPW * NW
NP = 10240
RPT = NP // NS

_mesh = plsc.VectorSubcoreMesh(
    core_axis_name="c", subcore_axis_name="s", num_cores=NC, num_subcores=NS)


CPW = EPW // CH


@functools.partial(
    pl.kernel,
    out_type=jax.ShapeDtypeStruct((NC, NP), jnp.float32),
    mesh=_mesh,
    scratch_types=[
        pltpu.VMEM((CPW, CH), jnp.int32),
        pltpu.VMEM((CH,), jnp.float32),
        pltpu.VMEM((RPT,), jnp.float32),
        pltpu.VMEM_SHARED((NP,), jnp.float32),
        pltpu.SemaphoreType.DMA,
    ],
)
def _deg_kernel(dst_hbm, out_hbm, dstall, ones_v, z_v, deg_sh, sem):
    c = lax.axis_index("c")
    s = lax.axis_index("s")
    w = s * NC + c

    def _ones_init(i, carry):
        ones_v[pl.ds(i * L, L)] = jnp.full((L,), 1.0, jnp.float32)
        return carry

    lax.fori_loop(0, CH // L, _ones_init, 0)

    def _zero_init(i, carry):
        z_v[pl.ds(i * L, L)] = jnp.zeros((L,), jnp.float32)
        return carry

    lax.fori_loop(0, RPT // L, _zero_init, 0)
    pltpu.sync_copy(z_v, deg_sh.at[pl.ds(s * RPT, RPT)])
    pltpu.sync_copy(dst_hbm.at[pl.ds(w * CPW, CPW)], dstall)
    plsc.subcore_barrier()

    def _issue(t, carry):
        pltpu.async_copy(ones_v, deg_sh.at[dstall.at[t]], sem, add=True)
        return carry

    lax.fori_loop(0, CPW, _issue, 0)

    def _drain(t, carry):
        pltpu.make_async_copy(ones_v, deg_sh.at[dstall.at[t]], sem).wait()
        return carry

    lax.fori_loop(0, CPW, _drain, 0)
    plsc.subcore_barrier()
    pltpu.sync_copy(deg_sh.at[pl.ds(s * RPT, RPT)],
                    out_hbm.at[c, pl.ds(s * RPT, RPT)])


@functools.partial(
    pl.kernel,
    out_type=jax.ShapeDtypeStruct((NC, NP, D), jnp.float32),
    mesh=_mesh,
    scratch_types=[
        pltpu.VMEM((HC, CH), jnp.int32),
        pltpu.VMEM((HC, CH), jnp.int32),
        pltpu.VMEM((CH, D), jnp.float32),
        pltpu.VMEM((CH, D), jnp.float32),
        pltpu.VMEM_SHARED((NP, D), jnp.float32),
        pltpu.SemaphoreType.DMA,
        pltpu.SemaphoreType.DMA,
        pltpu.SemaphoreType.DMA,
        pltpu.SemaphoreType.DMA,
    ],
)
def _scatter_kernel(y_hbm, src_hbm, dst_hbm, z_hbm, out_hbm,
                    srcall, dstall, rows0, rows1, acc_sh,
                    sg0, sg1, ss0, ss1):
    c = lax.axis_index("c")
    s = lax.axis_index("s")
    w = s * NC + c
    rows = (rows0, rows1)
    sg = (sg0, sg1)
    ss = (ss0, ss1)

    zd = pltpu.async_copy(z_hbm.at[pl.ds(s * RPT, RPT)],
                          acc_sh.at[pl.ds(s * RPT, RPT)], sg0)
    sd = pltpu.async_copy(src_hbm.at[pl.ds(w * CPW, HC)], srcall, sg1)
    dd = pltpu.async_copy(dst_hbm.at[pl.ds(w * CPW, HC)], dstall, ss0)
    zd.wait()
    sd.wait()
    dd.wait()

    def _issue_gather(t, b):
        pltpu.async_copy(y_hbm.at[srcall.at[t]], rows[b], sg[b])

    def _wait_gather(t, b):
        pltpu.make_async_copy(y_hbm.at[srcall.at[t]], rows[b], sg[b]).wait()

    def _issue_scatter(t, b):
        pltpu.async_copy(rows[b], acc_sh.at[dstall.at[t]], ss[b], add=True)

    def _wait_scatter(t, b):
        pltpu.make_async_copy(rows[b], acc_sh.at[dstall.at[t]], ss[b]).wait()

    def _run_half():
        _issue_gather(0, 0)
        _issue_gather(1, 1)
        _wait_gather(0, 0)
        _issue_scatter(0, 0)

        def _steady(i, carry):
            for u in (0, 1):
                t = 1 + i * 2 + u
                b = (1 + u) % 2
                _wait_scatter(t - 1, 1 - b)
                _issue_gather(t + 1, 1 - b)
                _wait_gather(t, b)
                _issue_scatter(t, b)
            return carry

        lax.fori_loop(0, (HC - 2) // 2, _steady, 0)
        t = HC - 1
        _wait_scatter(t - 1, 0)
        _wait_gather(t, 1)
        _issue_scatter(t, 1)
        _wait_scatter(t, 1)

    plsc.subcore_barrier()
    _run_half()
    base = w * CPW + HC
    pltpu.sync_copy(src_hbm.at[pl.ds(base, HC)], srcall)
    pltpu.sync_copy(dst_hbm.at[pl.ds(base, HC)], dstall)
    _run_half()
    plsc.subcore_barrier()
    pltpu.sync_copy(acc_sh.at[pl.ds(s * RPT, RPT)],
                    out_hbm.at[c, pl.ds(s * RPT, RPT)])


_BLK = 1000


def _l1a_body(x_ref, w_ref, u_ref):
    u_ref[...] = jnp.dot(x_ref[...], w_ref[...],
                         preferred_element_type=jnp.float32)


def _l1b_body(u_ref, d0_ref, d1_ref, y_ref, dinv_ref):
    dinv = lax.rsqrt(d0_ref[...] + d1_ref[...] + 1.0)
    y_ref[...] = u_ref[...] * dinv
    dinv_ref[...] = dinv


def _l2_body(p_ref, y_ref, dinv_ref, b_ref, w_ref, y2_ref):
    dinv = dinv_ref[...]
    h = jnp.maximum(dinv * (p_ref[0] + p_ref[1] + y_ref[...]) + b_ref[...], 0.0)
    y2_ref[...] = jnp.dot(h, w_ref[...],
                          preferred_element_type=jnp.float32) * dinv


def _l3_body(p_ref, y_ref, dinv_ref, b_ref, out_ref):
    out_ref[...] = jnp.maximum(
        dinv_ref[...] * (p_ref[0] + p_ref[1] + y_ref[...]) + b_ref[...], 0.0)


def _tc_l1a(x, W1):
    return pl.pallas_call(
        _l1a_body,
        grid=(N // _BLK,),
        in_specs=[
            pl.BlockSpec((_BLK, D), lambda i: (i, 0)),
            pl.BlockSpec((D, D), lambda i: (0, 0)),
        ],
        out_specs=pl.BlockSpec((_BLK, D), lambda i: (i, 0)),
        out_shape=jax.ShapeDtypeStruct((N, D), jnp.float32),
    )(x, W1)


def _tc_l1b(u, d0, d1):
    return pl.pallas_call(
        _l1b_body,
        grid=(N // _BLK,),
        in_specs=[
            pl.BlockSpec((_BLK, D), lambda i: (i, 0)),
            pl.BlockSpec((_BLK, 1), lambda i: (i, 0)),
            pl.BlockSpec((_BLK, 1), lambda i: (i, 0)),
        ],
        out_specs=[
            pl.BlockSpec((_BLK, D), lambda i: (i, 0)),
            pl.BlockSpec((_BLK, 1), lambda i: (i, 0)),
        ],
        out_shape=[
            jax.ShapeDtypeStruct((N, D), jnp.float32),
            jax.ShapeDtypeStruct((N, 1), jnp.float32),
        ],
    )(u, d0, d1)


def _tc_l2(p, y, dinv, b, W2):
    return pl.pallas_call(
        _l2_body,
        grid=(N // _BLK,),
        in_specs=[
            pl.BlockSpec((NC, _BLK, D), lambda i: (0, i, 0)),
            pl.BlockSpec((_BLK, D), lambda i: (i, 0)),
            pl.BlockSpec((_BLK, 1), lambda i: (i, 0)),
            pl.BlockSpec((1, D), lambda i: (0, 0)),
            pl.BlockSpec((D, D), lambda i: (0, 0)),
        ],
        out_specs=pl.BlockSpec((_BLK, D), lambda i: (i, 0)),
        out_shape=jax.ShapeDtypeStruct((N, D), jnp.float32),
    )(p, y, dinv, b, W2)


def _tc_l3(p, y, dinv, b):
    return pl.pallas_call(
        _l3_body,
        grid=(N // _BLK,),
        in_specs=[
            pl.BlockSpec((NC, _BLK, D), lambda i: (0, i, 0)),
            pl.BlockSpec((_BLK, D), lambda i: (i, 0)),
            pl.BlockSpec((_BLK, 1), lambda i: (i, 0)),
            pl.BlockSpec((1, D), lambda i: (0, 0)),
        ],
        out_specs=pl.BlockSpec((_BLK, D), lambda i: (i, 0)),
        out_shape=jax.ShapeDtypeStruct((N, D), jnp.float32),
    )(p, y, dinv, b)


def kernel(x, edge_index, W1, b1, W2, b2):
    src = edge_index[0].astype(jnp.int32)
    dst = edge_index[1].astype(jnp.int32)
    ppw = EPW - E // NW
    dummy_src = jnp.tile((jnp.arange(ppw, dtype=jnp.int32) * 41) % N, (NW, 1))
    dummy_dst = jnp.tile(N + (jnp.arange(ppw, dtype=jnp.int32) % (NP - N)),
                         (NW, 1))
    src_p = jnp.concatenate(
        [src.reshape(NW, E // NW), dummy_src], axis=1).reshape(EP)
    dst_p = jnp.concatenate(
        [dst.reshape(NW, E // NW), dummy_dst], axis=1).reshape(EP)

    src2d = src_p.reshape(EP // CH, CH)
    dst2d = dst_p.reshape(EP // CH, CH)

    degp = _deg_kernel(dst2d)
    d0 = degp[0, :N, None]
    d1 = degp[1, :N, None]

    zrows = jnp.zeros((NP, D), jnp.float32)

    u1 = _tc_l1a(x, W1)
    y1, dinv = _tc_l1b(u1, d0, d1)
    p1 = _scatter_kernel(y1, src2d, dst2d, zrows)
    y2 = _tc_l2(p1, y1, dinv, b1.reshape(1, D), W2)
    p2 = _scatter_kernel(y2, src2d, dst2d, zrows)
    return _tc_l3(p2, y2, dinv, b2.reshape(1, D))
